# baseline (speedup 1.0000x reference)
import functools

import jax
import jax.numpy as jnp
from jax import lax
from jax.experimental import pallas as pl
from jax.experimental.pallas import tpu as pltpu
from jax.experimental.pallas import tpu_sc as plsc

B = 5000
FANOUT = 10
D_IN = 128
D_H = 256
S = 200

BS = 2200
GSC = BS * FANOUT
C = 16
RC = C * FANOUT
NCHUNK = GSC // C
NW = 32
NB = 4

BF = jnp.bfloat16


def _sc_mean_body(x2_hbm, m2_hbm, *scratch):
    in_bufs = scratch[0:NB]
    out_bufs = scratch[NB:2 * NB]
    in_sems = scratch[2 * NB:3 * NB]
    out_sems = scratch[3 * NB:4 * NB]

    wid = lax.axis_index("s") * 2 + lax.axis_index("c")
    n_my = (NCHUNK - wid + NW - 1) // NW
    n_full = n_my // NB
    tail = n_my - n_full * NB

    def issue_in(k, b):
        pltpu.make_async_copy(
            x2_hbm.at[pl.ds((wid + k * NW) * RC, RC)], in_bufs[b], in_sems[b]
        ).start()

    def wait_in(b):
        pltpu.make_async_copy(
            x2_hbm.at[pl.ds(0, RC)], in_bufs[b], in_sems[b]
        ).wait()

    def issue_out(k, b):
        pltpu.make_async_copy(
            out_bufs[b], m2_hbm.at[pl.ds((wid + k * NW) * C, C)], out_sems[b]
        ).start()

    def wait_out(b):
        pltpu.make_async_copy(
            out_bufs[b], m2_hbm.at[pl.ds(0, C)], out_sems[b]
        ).wait()

    def compute(b):
        in_buf = in_bufs[b]
        out_buf = out_bufs[b]

        def g_body(g, carry):
            row0 = g * FANOUT
            for j in range(D_IN // 16):
                acc = in_buf[row0, pl.ds(j * 16, 16)]
                for r in range(1, FANOUT):
                    acc = acc + in_buf[row0 + r, pl.ds(j * 16, 16)]
                out_buf[g, pl.ds(j * 16, 16)] = acc
            return carry

        lax.fori_loop(0, C, g_body, 0)

    for b in range(NB):
        issue_in(b, b)

    def round_body(i, carry):
        for b in range(NB):
            k = i * NB + b
            wait_in(b)

            @pl.when(i > 0)
            def _():
                wait_out(b)

            compute(b)
            issue_out(k, b)
            nxt = (i + 1) * NB + b

            @pl.when(nxt < n_my)
            def _():
                issue_in(nxt, b)
        return carry

    lax.fori_loop(0, n_full, round_body, 0)

    for b in range(NB):
        k_tail = n_full * NB + b

        @pl.when(b < tail)
        def _():
            wait_in(b)
            wait_out(b)
            compute(b)
            issue_out(k_tail, b)

    for b in range(NB):
        wait_out(b)


_sc_segsum = functools.partial(
    pl.kernel,
    mesh=plsc.VectorSubcoreMesh(core_axis_name="c", subcore_axis_name="s"),
    out_type=jax.ShapeDtypeStruct((GSC, D_IN), jnp.float32),
    scratch_types=(
        [pltpu.VMEM((RC, D_IN), jnp.float32) for _ in range(NB)]
        + [pltpu.VMEM((C, D_IN), jnp.float32) for _ in range(NB)]
        + [pltpu.SemaphoreType.DMA for _ in range(2 * NB)]
    ),
)(_sc_mean_body)


def _dot(a, b):
    return jnp.dot(a, b, preferred_element_type=jnp.float32)


def _dense_tail(x0b, x1b, h1, sel1_ref, wa0_ref, ba0, ws0_ref,
                wa1_ref, ba1_ref, ws1_ref, out_ref):
    h1 = jnp.where(h1 >= 0, h1, 0.01 * h1)
    h1b = h1.astype(BF)
    sel1 = sel1_ref[...]
    mh1s = _dot(sel1, h1b)
    m1s = _dot(sel1, x1b)
    h0 = _dot(x0b, ws0_ref[...]) + _dot(m1s.astype(BF), wa0_ref[...]) + ba0
    h0 = jnp.where(h0 >= 0, h0, 0.01 * h0)
    out_ref[...] = (_dot(h0.astype(BF), ws1_ref[...])
                    + _dot(mh1s.astype(BF), wa1_ref[...]) + ba1_ref[...])


def _tc_fused_body(x0_ref, x1_ref, x2_ref, sel1_ref, wa0_ref, ba0_ref,
                   ws0_ref, wa1_ref, ba1_ref, ws1_ref, out_ref):
    ba0 = ba0_ref[...]
    m2 = x2_ref[...].reshape(S * FANOUT, FANOUT, D_IN).sum(axis=1)
    x1b = x1_ref[...].astype(BF)
    h1 = _dot(x1b, ws0_ref[...]) + _dot(m2.astype(BF), wa0_ref[...]) + ba0
    _dense_tail(x0_ref[...].astype(BF), x1b, h1, sel1_ref, wa0_ref, ba0,
                ws0_ref, wa1_ref, ba1_ref, ws1_ref, out_ref)


def _tc_dense_body(x0_ref, x1_ref, m2_ref, sel1_ref, wa0_ref, ba0_ref,
                   ws0_ref, wa1_ref, ba1_ref, ws1_ref, _acc_ref, out_ref):
    ba0 = ba0_ref[...]
    x1b = x1_ref[...].astype(BF)
    m2b = m2_ref[...].astype(BF)
    h1 = _dot(x1b, ws0_ref[...]) + _dot(m2b, wa0_ref[...]) + ba0
    _dense_tail(x0_ref[...].astype(BF), x1b, h1, sel1_ref, wa0_ref, ba0,
                ws0_ref, wa1_ref, ba1_ref, ws1_ref, out_ref)


def _weight_specs():
    full = lambda shape: pl.BlockSpec(shape, lambda i: (0,) * len(shape))
    return [
        full((S, S * FANOUT)),
        full((D_IN, D_H)),
        full((1, D_H)),
        full((D_IN, D_H)),
        full((D_H, D_H)),
        full((1, D_H)),
        full((D_H, D_H)),
    ]


def _tc_fused(x0, x1, x2, weights, boff, nblk):
    return pl.pallas_call(
        _tc_fused_body,
        grid=(nblk,),
        in_specs=[
            pl.BlockSpec((S, D_IN), lambda i: (i + boff, 0)),
            pl.BlockSpec((S * FANOUT, D_IN), lambda i: (i + boff, 0)),
            pl.BlockSpec((S * FANOUT * FANOUT, D_IN), lambda i: (i + boff, 0)),
        ] + _weight_specs(),
        out_specs=pl.BlockSpec((S, D_H), lambda i: (i + boff, 0)),
        out_shape=jax.ShapeDtypeStruct((B, D_H), jnp.float32),
        compiler_params=pltpu.CompilerParams(
            dimension_semantics=("arbitrary",),
        ),
    )(x0, x1, x2, *weights)


def _tc_dense(x0, x1, m2sum, weights, acc, nblk):
    return pl.pallas_call(
        _tc_dense_body,
        grid=(nblk,),
        in_specs=[
            pl.BlockSpec((S, D_IN), lambda i: (i, 0)),
            pl.BlockSpec((S * FANOUT, D_IN), lambda i: (i, 0)),
            pl.BlockSpec((S * FANOUT, D_IN), lambda i: (i, 0)),
        ] + _weight_specs() + [
            pl.BlockSpec(memory_space=pl.ANY),
        ],
        out_specs=pl.BlockSpec((S, D_H), lambda i: (i, 0)),
        out_shape=jax.ShapeDtypeStruct((B, D_H), jnp.float32),
        input_output_aliases={10: 0},
        compiler_params=pltpu.CompilerParams(
            dimension_semantics=("arbitrary",),
        ),
    )(x0, x1, m2sum, *weights, acc)


@jax.jit
def kernel(x0, x1, x2, W_agg0, b_agg0, W_self0, W_agg1, b_agg1, W_self1):
    scale = jnp.float32(1.0 / FANOUT)
    sel1 = (jnp.arange(S * FANOUT) // FANOUT
            == jnp.arange(S)[:, None]).astype(BF)
    weights = (sel1,
               (W_agg0 * scale).astype(BF), b_agg0.reshape(1, D_H),
               W_self0.astype(BF),
               (W_agg1 * scale).astype(BF), b_agg1.reshape(1, D_H),
               W_self1.astype(BF))

    m2sum = _sc_segsum(x2)
    out_hi = _tc_fused(x0, x1, x2, weights,
                       boff=BS // S, nblk=(B - BS) // S)
    return _tc_dense(x0, x1, m2sum, weights, out_hi, nblk=BS // S)


# speedup vs baseline: 1.0268x; 1.0268x over previous
import functools

import jax
import jax.numpy as jnp
from jax import lax
from jax.experimental import pallas as pl
from jax.experimental.pallas import tpu as pltpu
from jax.experimental.pallas import tpu_sc as plsc

B = 5000
FANOUT = 10
D_IN = 128
D_H = 256
S = 200

BS = 2600
GSC = BS * FANOUT
C = 16
RC = C * FANOUT
NCHUNK = GSC // C
NW = 32
NB = 4

BF = jnp.bfloat16


def _sc_mean_body(x2_hbm, m2_hbm, *scratch):
    in_bufs = scratch[0:NB]
    out_bufs = scratch[NB:2 * NB]
    in_sems = scratch[2 * NB:3 * NB]
    out_sems = scratch[3 * NB:4 * NB]

    wid = lax.axis_index("s") * 2 + lax.axis_index("c")
    n_my = (NCHUNK - wid + NW - 1) // NW
    n_full = n_my // NB
    tail = n_my - n_full * NB

    def issue_in(k, b):
        pltpu.make_async_copy(
            x2_hbm.at[pl.ds((wid + k * NW) * RC, RC)], in_bufs[b], in_sems[b]
        ).start()

    def wait_in(b):
        pltpu.make_async_copy(
            x2_hbm.at[pl.ds(0, RC)], in_bufs[b], in_sems[b]
        ).wait()

    def issue_out(k, b):
        pltpu.make_async_copy(
            out_bufs[b], m2_hbm.at[pl.ds((wid + k * NW) * C, C)], out_sems[b]
        ).start()

    def wait_out(b):
        pltpu.make_async_copy(
            out_bufs[b], m2_hbm.at[pl.ds(0, C)], out_sems[b]
        ).wait()

    def compute(b):
        in_buf = in_bufs[b]
        out_buf = out_bufs[b]

        def g_body(g, carry):
            row0 = g * FANOUT
            for j in range(D_IN // 16):
                acc = in_buf[row0, pl.ds(j * 16, 16)]
                for r in range(1, FANOUT):
                    acc = acc + in_buf[row0 + r, pl.ds(j * 16, 16)]
                out_buf[g, pl.ds(j * 16, 16)] = acc
            return carry

        lax.fori_loop(0, C, g_body, 0)

    for b in range(NB):
        issue_in(b, b)

    def round_body(i, carry):
        for b in range(NB):
            k = i * NB + b
            wait_in(b)

            @pl.when(i > 0)
            def _():
                wait_out(b)

            compute(b)
            issue_out(k, b)
            nxt = (i + 1) * NB + b

            @pl.when(nxt < n_my)
            def _():
                issue_in(nxt, b)
        return carry

    lax.fori_loop(0, n_full, round_body, 0)

    for b in range(NB):
        k_tail = n_full * NB + b

        @pl.when(b < tail)
        def _():
            wait_in(b)
            wait_out(b)
            compute(b)
            issue_out(k_tail, b)

    for b in range(NB):
        wait_out(b)


_sc_segsum = functools.partial(
    pl.kernel,
    mesh=plsc.VectorSubcoreMesh(core_axis_name="c", subcore_axis_name="s"),
    out_type=jax.ShapeDtypeStruct((GSC, D_IN), jnp.float32),
    scratch_types=(
        [pltpu.VMEM((RC, D_IN), jnp.float32) for _ in range(NB)]
        + [pltpu.VMEM((C, D_IN), jnp.float32) for _ in range(NB)]
        + [pltpu.SemaphoreType.DMA for _ in range(2 * NB)]
    ),
)(_sc_mean_body)


def _dot(a, b):
    return jnp.dot(a, b, preferred_element_type=jnp.float32)


def _dense_tail(x0b, x1b, h1, sel1_ref, wa0_ref, ba0, ws0_ref,
                wa1_ref, ba1_ref, ws1_ref, out_ref):
    h1 = jnp.where(h1 >= 0, h1, 0.01 * h1)
    h1b = h1.astype(BF)
    sel1 = sel1_ref[...]
    mh1s = _dot(sel1, h1b)
    m1s = _dot(sel1, x1b)
    h0 = _dot(x0b, ws0_ref[...]) + _dot(m1s.astype(BF), wa0_ref[...]) + ba0
    h0 = jnp.where(h0 >= 0, h0, 0.01 * h0)
    out_ref[...] = (_dot(h0.astype(BF), ws1_ref[...])
                    + _dot(mh1s.astype(BF), wa1_ref[...]) + ba1_ref[...])


def _tc_fused_body(x0_ref, x1_ref, x2_ref, sel1_ref, wa0_ref, ba0_ref,
                   ws0_ref, wa1_ref, ba1_ref, ws1_ref, out_ref):
    ba0 = ba0_ref[...]
    m2 = x2_ref[...].reshape(S * FANOUT, FANOUT, D_IN).sum(axis=1)
    x1b = x1_ref[...].astype(BF)
    h1 = _dot(x1b, ws0_ref[...]) + _dot(m2.astype(BF), wa0_ref[...]) + ba0
    _dense_tail(x0_ref[...].astype(BF), x1b, h1, sel1_ref, wa0_ref, ba0,
                ws0_ref, wa1_ref, ba1_ref, ws1_ref, out_ref)


def _tc_dense_body(x0_ref, x1_ref, m2_ref, sel1_ref, wa0_ref, ba0_ref,
                   ws0_ref, wa1_ref, ba1_ref, ws1_ref, _acc_ref, out_ref):
    ba0 = ba0_ref[...]
    x1b = x1_ref[...].astype(BF)
    m2b = m2_ref[...].astype(BF)
    h1 = _dot(x1b, ws0_ref[...]) + _dot(m2b, wa0_ref[...]) + ba0
    _dense_tail(x0_ref[...].astype(BF), x1b, h1, sel1_ref, wa0_ref, ba0,
                ws0_ref, wa1_ref, ba1_ref, ws1_ref, out_ref)


def _weight_specs():
    full = lambda shape: pl.BlockSpec(shape, lambda i: (0,) * len(shape))
    return [
        full((S, S * FANOUT)),
        full((D_IN, D_H)),
        full((1, D_H)),
        full((D_IN, D_H)),
        full((D_H, D_H)),
        full((1, D_H)),
        full((D_H, D_H)),
    ]


def _tc_fused(x0, x1, x2, weights, boff, nblk):
    return pl.pallas_call(
        _tc_fused_body,
        grid=(nblk,),
        in_specs=[
            pl.BlockSpec((S, D_IN), lambda i: (i + boff, 0)),
            pl.BlockSpec((S * FANOUT, D_IN), lambda i: (i + boff, 0)),
            pl.BlockSpec((S * FANOUT * FANOUT, D_IN), lambda i: (i + boff, 0)),
        ] + _weight_specs(),
        out_specs=pl.BlockSpec((S, D_H), lambda i: (i + boff, 0)),
        out_shape=jax.ShapeDtypeStruct((B, D_H), jnp.float32),
        compiler_params=pltpu.CompilerParams(
            dimension_semantics=("arbitrary",),
        ),
    )(x0, x1, x2, *weights)


def _tc_dense(x0, x1, m2sum, weights, acc, nblk):
    return pl.pallas_call(
        _tc_dense_body,
        grid=(nblk,),
        in_specs=[
            pl.BlockSpec((S, D_IN), lambda i: (i, 0)),
            pl.BlockSpec((S * FANOUT, D_IN), lambda i: (i, 0)),
            pl.BlockSpec((S * FANOUT, D_IN), lambda i: (i, 0)),
        ] + _weight_specs() + [
            pl.BlockSpec(memory_space=pl.ANY),
        ],
        out_specs=pl.BlockSpec((S, D_H), lambda i: (i, 0)),
        out_shape=jax.ShapeDtypeStruct((B, D_H), jnp.float32),
        input_output_aliases={10: 0},
        compiler_params=pltpu.CompilerParams(
            dimension_semantics=("arbitrary",),
        ),
    )(x0, x1, m2sum, *weights, acc)


@jax.jit
def kernel(x0, x1, x2, W_agg0, b_agg0, W_self0, W_agg1, b_agg1, W_self1):
    scale = jnp.float32(1.0 / FANOUT)
    sel1 = (jnp.arange(S * FANOUT) // FANOUT
            == jnp.arange(S)[:, None]).astype(BF)
    weights = (sel1,
               (W_agg0 * scale).astype(BF), b_agg0.reshape(1, D_H),
               W_self0.astype(BF),
               (W_agg1 * scale).astype(BF), b_agg1.reshape(1, D_H),
               W_self1.astype(BF))

    m2sum = _sc_segsum(x2)
    out_hi = _tc_fused(x0, x1, x2, weights,
                       boff=BS // S, nblk=(B - BS) // S)
    return _tc_dense(x0, x1, m2sum, weights, out_hi, nblk=BS // S)


# speedup vs baseline: 1.0361x; 1.0090x over previous
import functools

import jax
import jax.numpy as jnp
from jax import lax
from jax.experimental import pallas as pl
from jax.experimental.pallas import tpu as pltpu
from jax.experimental.pallas import tpu_sc as plsc

B = 5000
FANOUT = 10
D_IN = 128
D_H = 256
S = 200

BS = 2400
GSC = BS * FANOUT
C = 16
RC = C * FANOUT
NCHUNK = GSC // C
NW = 32
NB = 4

BF = jnp.bfloat16


def _sc_mean_body(x2_hbm, m2_hbm, *scratch):
    in_bufs = scratch[0:NB]
    out_bufs = scratch[NB:2 * NB]
    in_sems = scratch[2 * NB:3 * NB]
    out_sems = scratch[3 * NB:4 * NB]

    wid = lax.axis_index("s") * 2 + lax.axis_index("c")
    n_my = (NCHUNK - wid + NW - 1) // NW
    n_full = n_my // NB
    tail = n_my - n_full * NB

    def issue_in(k, b):
        pltpu.make_async_copy(
            x2_hbm.at[pl.ds((wid + k * NW) * RC, RC)], in_bufs[b], in_sems[b]
        ).start()

    def wait_in(b):
        pltpu.make_async_copy(
            x2_hbm.at[pl.ds(0, RC)], in_bufs[b], in_sems[b]
        ).wait()

    def issue_out(k, b):
        pltpu.make_async_copy(
            out_bufs[b], m2_hbm.at[pl.ds((wid + k * NW) * C, C)], out_sems[b]
        ).start()

    def wait_out(b):
        pltpu.make_async_copy(
            out_bufs[b], m2_hbm.at[pl.ds(0, C)], out_sems[b]
        ).wait()

    def compute(b):
        in_buf = in_bufs[b]
        out_buf = out_bufs[b]

        def g_body(g, carry):
            row0 = g * FANOUT
            for j in range(D_IN // 16):
                acc = in_buf[row0, pl.ds(j * 16, 16)]
                for r in range(1, FANOUT):
                    acc = acc + in_buf[row0 + r, pl.ds(j * 16, 16)]
                out_buf[g, pl.ds(j * 16, 16)] = acc
            return carry

        lax.fori_loop(0, C, g_body, 0)

    for b in range(NB):
        issue_in(b, b)

    def round_body(i, carry):
        for b in range(NB):
            k = i * NB + b
            wait_in(b)

            @pl.when(i > 0)
            def _():
                wait_out(b)

            compute(b)
            issue_out(k, b)
            nxt = (i + 1) * NB + b

            @pl.when(nxt < n_my)
            def _():
                issue_in(nxt, b)
        return carry

    lax.fori_loop(0, n_full, round_body, 0)

    for b in range(NB):
        k_tail = n_full * NB + b

        @pl.when(b < tail)
        def _():
            wait_in(b)
            wait_out(b)
            compute(b)
            issue_out(k_tail, b)

    for b in range(NB):
        wait_out(b)


_sc_segsum = functools.partial(
    pl.kernel,
    mesh=plsc.VectorSubcoreMesh(core_axis_name="c", subcore_axis_name="s"),
    out_type=jax.ShapeDtypeStruct((GSC, D_IN), jnp.float32),
    scratch_types=(
        [pltpu.VMEM((RC, D_IN), jnp.float32) for _ in range(NB)]
        + [pltpu.VMEM((C, D_IN), jnp.float32) for _ in range(NB)]
        + [pltpu.SemaphoreType.DMA for _ in range(2 * NB)]
    ),
)(_sc_mean_body)


def _dot(a, b):
    return jnp.dot(a, b, preferred_element_type=jnp.float32)


def _dense_tail(x0b, x1b, h1, sel1_ref, wa0_ref, ba0, ws0_ref,
                wa1_ref, ba1_ref, ws1_ref, out_ref):
    h1 = jnp.where(h1 >= 0, h1, 0.01 * h1)
    h1b = h1.astype(BF)
    sel1 = sel1_ref[...]
    mh1s = _dot(sel1, h1b)
    m1s = _dot(sel1, x1b)
    h0 = _dot(x0b, ws0_ref[...]) + _dot(m1s.astype(BF), wa0_ref[...]) + ba0
    h0 = jnp.where(h0 >= 0, h0, 0.01 * h0)
    out_ref[...] = (_dot(h0.astype(BF), ws1_ref[...])
                    + _dot(mh1s.astype(BF), wa1_ref[...]) + ba1_ref[...])


def _tc_fused_body(x0_ref, x1_ref, x2_ref, sel1_ref, wa0_ref, ba0_ref,
                   ws0_ref, wa1_ref, ba1_ref, ws1_ref, out_ref):
    ba0 = ba0_ref[...]
    m2 = x2_ref[...].reshape(S * FANOUT, FANOUT, D_IN).sum(axis=1)
    x1b = x1_ref[...].astype(BF)
    h1 = _dot(x1b, ws0_ref[...]) + _dot(m2.astype(BF), wa0_ref[...]) + ba0
    _dense_tail(x0_ref[...].astype(BF), x1b, h1, sel1_ref, wa0_ref, ba0,
                ws0_ref, wa1_ref, ba1_ref, ws1_ref, out_ref)


def _tc_dense_body(x0_ref, x1_ref, m2_ref, sel1_ref, wa0_ref, ba0_ref,
                   ws0_ref, wa1_ref, ba1_ref, ws1_ref, _acc_ref, out_ref):
    ba0 = ba0_ref[...]
    x1b = x1_ref[...].astype(BF)
    m2b = m2_ref[...].astype(BF)
    h1 = _dot(x1b, ws0_ref[...]) + _dot(m2b, wa0_ref[...]) + ba0
    _dense_tail(x0_ref[...].astype(BF), x1b, h1, sel1_ref, wa0_ref, ba0,
                ws0_ref, wa1_ref, ba1_ref, ws1_ref, out_ref)


def _weight_specs():
    full = lambda shape: pl.BlockSpec(shape, lambda i: (0,) * len(shape))
    return [
        full((S, S * FANOUT)),
        full((D_IN, D_H)),
        full((1, D_H)),
        full((D_IN, D_H)),
        full((D_H, D_H)),
        full((1, D_H)),
        full((D_H, D_H)),
    ]


def _tc_fused(x0, x1, x2, weights, boff, nblk):
    return pl.pallas_call(
        _tc_fused_body,
        grid=(nblk,),
        in_specs=[
            pl.BlockSpec((S, D_IN), lambda i: (i + boff, 0)),
            pl.BlockSpec((S * FANOUT, D_IN), lambda i: (i + boff, 0)),
            pl.BlockSpec((S * FANOUT * FANOUT, D_IN), lambda i: (i + boff, 0)),
        ] + _weight_specs(),
        out_specs=pl.BlockSpec((S, D_H), lambda i: (i + boff, 0)),
        out_shape=jax.ShapeDtypeStruct((B, D_H), jnp.float32),
        compiler_params=pltpu.CompilerParams(
            dimension_semantics=("arbitrary",),
        ),
    )(x0, x1, x2, *weights)


def _tc_dense(x0, x1, m2sum, weights, acc, nblk):
    return pl.pallas_call(
        _tc_dense_body,
        grid=(nblk,),
        in_specs=[
            pl.BlockSpec((S, D_IN), lambda i: (i, 0)),
            pl.BlockSpec((S * FANOUT, D_IN), lambda i: (i, 0)),
            pl.BlockSpec((S * FANOUT, D_IN), lambda i: (i, 0)),
        ] + _weight_specs() + [
            pl.BlockSpec(memory_space=pl.ANY),
        ],
        out_specs=pl.BlockSpec((S, D_H), lambda i: (i, 0)),
        out_shape=jax.ShapeDtypeStruct((B, D_H), jnp.float32),
        input_output_aliases={10: 0},
        compiler_params=pltpu.CompilerParams(
            dimension_semantics=("arbitrary",),
        ),
    )(x0, x1, m2sum, *weights, acc)


@jax.jit
def kernel(x0, x1, x2, W_agg0, b_agg0, W_self0, W_agg1, b_agg1, W_self1):
    scale = jnp.float32(1.0 / FANOUT)
    sel1 = (jnp.arange(S * FANOUT) // FANOUT
            == jnp.arange(S)[:, None]).astype(BF)
    weights = (sel1,
               (W_agg0 * scale).astype(BF), b_agg0.reshape(1, D_H),
               W_self0.astype(BF),
               (W_agg1 * scale).astype(BF), b_agg1.reshape(1, D_H),
               W_self1.astype(BF))

    m2sum = _sc_segsum(x2)
    out_hi = _tc_fused(x0, x1, x2, weights,
                       boff=BS // S, nblk=(B - BS) // S)
    return _tc_dense(x0, x1, m2sum, weights, out_hi, nblk=BS // S)
